# baseline (speedup 1.0000x reference)
import functools

import jax
import jax.numpy as jnp
import numpy as np
from jax import lax
from jax.experimental import pallas as pl
from jax.experimental.pallas import tpu as pltpu
from jax.experimental.pallas import tpu_sc as plsc

NUM_BINS = 10
MIN_CLAMP = 0.01
MAX_CLAMP = 100.0
BATCH = 16384
NUM_CLASSES = 1000

_EDGE_BITS = np.array(
    [0x00000000, 0x3DCCCCCD, 0x3E4CCCCD, 0x3E99999A, 0x3ECCCCCD, 0x3F000000,
     0x3F19999A, 0x3F333333, 0x3F4CCCCD, 0x3F666667, 0x3F800000],
    dtype=np.uint32)
_EDGES = tuple(_EDGE_BITS.view(np.float32).tolist())

_NC, _NS = 2, 16
_NW = _NC * _NS
_LANES = 16
_ROWS_PER_W = BATCH // _NW
_GROUPS = _ROWS_PER_W // _LANES

_TC_BLOCK_COLS = 2048
_TC_GRID = BATCH // _TC_BLOCK_COLS


_ZROWS = 1024
_ZDMAS_PER_STEP = BATCH // _ZROWS // _TC_GRID


def _tc_colmax_kernel(prob_ref, out_ref, calib_hbm, zbuf, zsem):
    i = pl.program_id(0)

    @pl.when(i == 0)
    def _():
        zbuf[...] = jnp.zeros((_ZROWS, NUM_CLASSES), jnp.float32)

    def zdma(block_idx):
        return pltpu.make_async_copy(
            zbuf, calib_hbm.at[pl.ds(block_idx * _ZROWS, _ZROWS), :], zsem)

    for k in range(_ZDMAS_PER_STEP):
        zdma(i * _ZDMAS_PER_STEP + k).start()

    m = jnp.max(prob_ref[...], axis=0)
    out_ref[...] = m.reshape(_TC_BLOCK_COLS // 128, 128)

    @pl.when(i == _TC_GRID - 1)
    def _():
        for _k in range(_TC_GRID * _ZDMAS_PER_STEP):
            zdma(0).wait()


def _sc_bin_kernel(mx_hbm, params_hbm, out_hbm, mx_v, params_v, out_v):
    cid = lax.axis_index("c")
    sid = lax.axis_index("s")
    wid = cid * _NS + sid
    base = wid * _ROWS_PER_W

    pltpu.sync_copy(params_hbm, params_v)
    pltpu.sync_copy(mx_hbm.at[pl.ds(base, _ROWS_PER_W)], mx_v)

    for k in range(_GROUPS):
        m = mx_v[pl.ds(k * _LANES, _LANES)]
        cnt = jnp.zeros((_LANES,), jnp.int32)
        for e in _EDGES:
            cnt = cnt + jnp.where(m > jnp.float32(e),
                                  jnp.int32(1), jnp.int32(0))
        bin_idx = jnp.clip(cnt - 1, 0, NUM_BINS - 1)
        bv = plsc.load_gather(params_v, [bin_idx])
        temp = jnp.clip(bv * bv, MIN_CLAMP, MAX_CLAMP)
        out_v[pl.ds(k * _LANES, _LANES)] = m * (1.0 / temp)

    pltpu.sync_copy(out_v, out_hbm.at[pl.ds(base, _ROWS_PER_W)])


@jax.jit
def _run(prob, params16):
    maxes, calibrated = pl.pallas_call(
        _tc_colmax_kernel,
        grid=(_TC_GRID,),
        in_specs=[pl.BlockSpec((NUM_CLASSES, _TC_BLOCK_COLS),
                               lambda i: (0, i))],
        out_specs=[
            pl.BlockSpec((_TC_BLOCK_COLS // 128, 128), lambda i: (i, 0)),
            pl.BlockSpec(memory_space=pl.ANY),
        ],
        out_shape=[
            jax.ShapeDtypeStruct((BATCH // 128, 128), jnp.float32),
            jax.ShapeDtypeStruct((BATCH, NUM_CLASSES), jnp.float32),
        ],
        scratch_shapes=[
            pltpu.VMEM((_ZROWS, NUM_CLASSES), jnp.float32),
            pltpu.SemaphoreType.DMA,
        ],
    )(prob.T)

    mesh = plsc.VectorSubcoreMesh(core_axis_name="c", subcore_axis_name="s",
                                  num_cores=_NC, num_subcores=_NS)
    f = pl.kernel(
        _sc_bin_kernel,
        out_type=jax.ShapeDtypeStruct((BATCH,), jnp.float32),
        mesh=mesh,
        scratch_types=[
            pltpu.VMEM((_ROWS_PER_W,), jnp.float32),
            pltpu.VMEM((_LANES,), jnp.float32),
            pltpu.VMEM((_ROWS_PER_W,), jnp.float32),
        ],
        compiler_params=pltpu.CompilerParams(needs_layout_passes=False),
    )
    output = f(maxes.reshape(BATCH), params16)
    return output, calibrated


def kernel(probabilities, bin_params):
    params16 = jnp.concatenate(
        [bin_params, jnp.zeros((_LANES - NUM_BINS,), jnp.float32)])
    output, calibrated = _run(probabilities, params16)
    return (output, calibrated)


# speedup vs baseline: 1.8211x; 1.8211x over previous
import functools

import jax
import jax.numpy as jnp
import numpy as np
from jax import lax
from jax.experimental import pallas as pl
from jax.experimental.pallas import tpu as pltpu
from jax.experimental.pallas import tpu_sc as plsc

NUM_BINS = 10
MIN_CLAMP = 0.01
MAX_CLAMP = 100.0
BATCH = 16384
NUM_CLASSES = 1000

_EDGE_BITS = np.array(
    [0x00000000, 0x3DCCCCCD, 0x3E4CCCCD, 0x3E99999A, 0x3ECCCCCD, 0x3F000000,
     0x3F19999A, 0x3F333333, 0x3F4CCCCD, 0x3F666667, 0x3F800000],
    dtype=np.uint32)
_EDGES = tuple(_EDGE_BITS.view(np.float32).tolist())

_NC, _NS = 2, 16
_NW = _NC * _NS
_LANES = 16
_ROWS_PER_W = BATCH // _NW
_GROUPS = _ROWS_PER_W // _LANES

_TC_BLOCK_COLS = 2048
_TC_GRID = BATCH // _TC_BLOCK_COLS


def _tc_colmax_kernel(prob_ref, out_ref):
    m = jnp.max(prob_ref[...], axis=0)
    out_ref[...] = m.reshape(_TC_BLOCK_COLS // 128, 128)


def _sc_bin_kernel(mx_hbm, params_hbm, out_hbm, mx_v, params_v, out_v):
    cid = lax.axis_index("c")
    sid = lax.axis_index("s")
    wid = cid * _NS + sid
    base = wid * _ROWS_PER_W

    pltpu.sync_copy(params_hbm, params_v)
    pltpu.sync_copy(mx_hbm.at[pl.ds(base, _ROWS_PER_W)], mx_v)

    for k in range(_GROUPS):
        m = mx_v[pl.ds(k * _LANES, _LANES)]
        cnt = jnp.zeros((_LANES,), jnp.int32)
        for e in _EDGES:
            cnt = cnt + jnp.where(m > jnp.float32(e),
                                  jnp.int32(1), jnp.int32(0))
        bin_idx = jnp.clip(cnt - 1, 0, NUM_BINS - 1)
        bv = plsc.load_gather(params_v, [bin_idx])
        temp = jnp.clip(bv * bv, MIN_CLAMP, MAX_CLAMP)
        out_v[pl.ds(k * _LANES, _LANES)] = m * (1.0 / temp)

    pltpu.sync_copy(out_v, out_hbm.at[pl.ds(base, _ROWS_PER_W)])


@jax.jit
def _run(prob, params16):
    maxes = pl.pallas_call(
        _tc_colmax_kernel,
        grid=(_TC_GRID,),
        in_specs=[pl.BlockSpec((NUM_CLASSES, _TC_BLOCK_COLS),
                               lambda i: (0, i))],
        out_specs=pl.BlockSpec((_TC_BLOCK_COLS // 128, 128),
                               lambda i: (i, 0)),
        out_shape=jax.ShapeDtypeStruct((BATCH // 128, 128), jnp.float32),
    )(prob.T)

    mesh = plsc.VectorSubcoreMesh(core_axis_name="c", subcore_axis_name="s",
                                  num_cores=_NC, num_subcores=_NS)
    f = pl.kernel(
        _sc_bin_kernel,
        out_type=jax.ShapeDtypeStruct((BATCH,), jnp.float32),
        mesh=mesh,
        scratch_types=[
            pltpu.VMEM((_ROWS_PER_W,), jnp.float32),
            pltpu.VMEM((_LANES,), jnp.float32),
            pltpu.VMEM((_ROWS_PER_W,), jnp.float32),
        ],
        compiler_params=pltpu.CompilerParams(needs_layout_passes=False),
    )
    output = f(maxes.reshape(BATCH), params16)
    calibrated = jnp.broadcast_to(
        (params16[0] * jnp.float32(0.0)).reshape(1, 1),
        (BATCH, NUM_CLASSES))
    return output, calibrated


def kernel(probabilities, bin_params):
    params16 = jnp.concatenate(
        [bin_params, jnp.zeros((_LANES - NUM_BINS,), jnp.float32)])
    output, calibrated = _run(probabilities, params16)
    return (output, calibrated)
